# baseline (speedup 1.0000x reference)
import functools

import jax
import jax.numpy as jnp
from jax import lax
from jax.experimental import pallas as pl
from jax.experimental.pallas import tpu as pltpu
from jax.experimental.pallas import tpu_sc as plsc

NUM_CORES = 2
NUM_SUBCORES = 16
NUM_WORKERS = NUM_CORES * NUM_SUBCORES

EMBED_DIM = 64
PROJ_DIM = 256
LANES = 16

TC_BLK = 4096
HALF = TC_BLK // 2

IDX_W = 128
GATHERS_PER_ITER = 4
CHUNK = IDX_W * GATHERS_PER_ITER
CHUNKS_PER_BLK = TC_BLK // CHUNK


VOCAB = 1000000
TILE_COLS = VOCAB // 128


def _sc_table_prep(tt, tail2):
  mesh = plsc.VectorSubcoreMesh(core_axis_name="c", subcore_axis_name="s")
  per_worker = TILE_COLS // NUM_WORKERS + 1

  @functools.partial(
      pl.kernel,
      mesh=mesh,
      out_type=jax.ShapeDtypeStruct((VOCAB // 2, 128), jnp.float32),
      compiler_params=pltpu.CompilerParams(use_tc_tiling_on_sc=True,
                                           needs_layout_passes=False),
      scratch_types=[
          pltpu.VMEM((64, 128), jnp.float32),
          pltpu.VMEM((64, 128), jnp.float32),
      ],
  )
  def prep_kernel(tt_hbm, tail_hbm, out_hbm, in_v, out_v):
    wid = lax.axis_index("s") * NUM_CORES + lax.axis_index("c")
    iota = jax.lax.iota(jnp.int32, LANES)

    def body(i, carry):
      c = wid + NUM_WORKERS * i

      @pl.when(c < TILE_COLS)
      def _():
        pltpu.sync_copy(tt_hbm.at[:, pl.ds(c * 128, 128)], in_v)

        def row(rp, carry2):
          for h in range(2):
            col = 2 * rp + h
            for g in range(GATHERS_PER_ITER):
              vals = plsc.load_gather(
                  in_v, [LANES * g + iota, jnp.full((LANES,), col, jnp.int32)])
              plsc.store_scatter(
                  out_v,
                  [jnp.full((LANES,), rp, jnp.int32),
                   64 * h + LANES * g + iota],
                  vals)
          return carry2

        lax.fori_loop(0, 64, row, 0)
        pltpu.sync_copy(out_v, out_hbm.at[pl.ds(c * 64, 64)])

      return carry

    lax.fori_loop(0, per_worker, body, 0)

    @pl.when(wid == 0)
    def _():
      pltpu.sync_copy(tail_hbm, in_v.at[pl.ds(0, 32)])
      pltpu.sync_copy(in_v.at[pl.ds(0, 32)],
                      out_hbm.at[pl.ds((TILE_COLS * 128) // 2, 32)])

  return prep_kernel(tt, tail2)


def _sc_gather(ids1d, table, n_rows):
  per_worker = n_rows // NUM_WORKERS
  iters = per_worker // CHUNK
  blocks_per_worker = per_worker // TC_BLK

  mesh = plsc.VectorSubcoreMesh(core_axis_name="c", subcore_axis_name="s")

  @functools.partial(
      pl.kernel,
      mesh=mesh,
      out_type=jax.ShapeDtypeStruct((n_rows, EMBED_DIM), jnp.float32),
      compiler_params=pltpu.CompilerParams(use_tc_tiling_on_sc=False, needs_layout_passes=False),
      scratch_types=[
          pltpu.VMEM((CHUNK,), jnp.int32),
          [pltpu.VMEM((IDX_W,), jnp.int32)] * GATHERS_PER_ITER,
          pltpu.VMEM((CHUNK, EMBED_DIM), jnp.float32),
          pltpu.SemaphoreType.DMA,
      ],
  )
  def gather_kernel(ids_hbm, table_hbm, emb_hbm, raw_v, idx_vs, rows_v, sem):
    wid = lax.axis_index("s") * NUM_CORES + lax.axis_index("c")
    blk0 = wid * blocks_per_worker
    row0 = wid * per_worker

    def body(t, carry):
      blk = blk0 + t // CHUNKS_PER_BLK
      sub = t % CHUNKS_PER_BLK
      l_off = blk * TC_BLK + (CHUNK // 2) * sub
      pltpu.sync_copy(ids_hbm.at[pl.ds(l_off, CHUNK // 2)],
                      raw_v.at[pl.ds(0, CHUNK // 2)])
      pltpu.sync_copy(ids_hbm.at[pl.ds(l_off + HALF, CHUNK // 2)],
                      raw_v.at[pl.ds(CHUNK // 2, CHUNK // 2)])
      lane2 = 2 * jnp.arange(LANES, dtype=jnp.int32)
      for v in range(2 * LANES):
        vals = raw_v[pl.ds(LANES * v, LANES)]
        vv = v % LANES
        dst = lane2 + (32 * (vv % 4) + (0 if v < LANES else 1))
        plsc.store_scatter(idx_vs[vv // 4], [dst], vals)
      copies = []
      for j in range(GATHERS_PER_ITER):
        copies.append(
            pltpu.async_copy(
                table_hbm.at[idx_vs[j]],
                rows_v.at[pl.ds(j * IDX_W, IDX_W)],
                sem))
      for c in copies:
        c.wait()
      pltpu.sync_copy(rows_v, emb_hbm.at[pl.ds(row0 + t * CHUNK, CHUNK)])
      return carry

    lax.fori_loop(0, iters, body, 0)

  return gather_kernel(ids1d, table)


NUM_CHUNKS = 5


def _tc_project_chunk(emb2, wt, prev_out, n_rows, chunk, chunk_rows):
  grid = (chunk_rows // TC_BLK,)
  blk0 = chunk * (chunk_rows // TC_BLK)

  def matmul_kernel(emb_ref, wt_ref, *refs):
    out_ref = refs[-1]
    blk = emb_ref[...]
    out_ref[0:HALF, :] = jnp.dot(
        blk[:, 0:EMBED_DIM], wt_ref[...], preferred_element_type=jnp.float32)
    out_ref[HALF:TC_BLK, :] = jnp.dot(
        blk[:, EMBED_DIM:2 * EMBED_DIM], wt_ref[...],
        preferred_element_type=jnp.float32)

  in_specs = [
      pl.BlockSpec((HALF, 2 * EMBED_DIM), lambda i: (i, 0)),
      pl.BlockSpec((EMBED_DIM, PROJ_DIM), lambda i: (0, 0)),
  ]
  args = [emb2, wt]
  aliases = {}
  if prev_out is not None:
    in_specs.append(pl.BlockSpec(memory_space=pl.ANY))
    args.append(prev_out)
    aliases = {2: 0}
  return pl.pallas_call(
      matmul_kernel,
      grid=grid,
      in_specs=in_specs,
      out_specs=pl.BlockSpec((TC_BLK, PROJ_DIM), lambda i: (blk0 + i, 0)),
      out_shape=jax.ShapeDtypeStruct((n_rows, PROJ_DIM), jnp.float32),
      input_output_aliases=aliases,
  )(*args)


@jax.jit
def _run(token_ids, embed_table, proj_weight):
  b, l = token_ids.shape
  n = b * l
  chunk_rows = n // NUM_CHUNKS
  ids1d = token_ids.astype(jnp.int32).reshape(n)
  wt = proj_weight.T
  tail2 = lax.slice(embed_table, (TILE_COLS * 128, 0), (VOCAB, EMBED_DIM))
  tail2 = tail2.reshape(32, 128)
  table2 = _sc_table_prep(embed_table.T, tail2)
  table_lin = table2.reshape(VOCAB, EMBED_DIM)
  out = None
  for c in range(NUM_CHUNKS):
    ids_c = lax.slice(ids1d, (c * chunk_rows,), ((c + 1) * chunk_rows,))
    emb = _sc_gather(ids_c, table_lin, chunk_rows)
    emb2 = emb.reshape(chunk_rows // 2, 2 * EMBED_DIM)
    out = _tc_project_chunk(emb2, wt, out, n, c, chunk_rows)
  return out.reshape(b, l, PROJ_DIM)


def kernel(token_ids, embed_table, proj_weight):
  return _run(token_ids, embed_table, proj_weight)


# speedup vs baseline: 1.4920x; 1.4920x over previous
import functools

import jax
import jax.numpy as jnp
from jax import lax
from jax.experimental import pallas as pl
from jax.experimental.pallas import tpu as pltpu
from jax.experimental.pallas import tpu_sc as plsc

NUM_CORES = 2
NUM_SUBCORES = 16
NUM_WORKERS = NUM_CORES * NUM_SUBCORES

EMBED_DIM = 64
PROJ_DIM = 256
LANES = 16

TC_BLK = 4096
HALF = TC_BLK // 2

IDX_W = 128
GATHERS_PER_ITER = 4
CHUNK = IDX_W * GATHERS_PER_ITER
CHUNKS_PER_BLK = TC_BLK // CHUNK


def _sc_gather(ids1d, table, n_rows):
  per_worker = n_rows // NUM_WORKERS
  iters = per_worker // CHUNK
  blocks_per_worker = per_worker // TC_BLK

  mesh = plsc.VectorSubcoreMesh(core_axis_name="c", subcore_axis_name="s")

  @functools.partial(
      pl.kernel,
      mesh=mesh,
      out_type=jax.ShapeDtypeStruct((n_rows, EMBED_DIM), jnp.float32),
      compiler_params=pltpu.CompilerParams(use_tc_tiling_on_sc=False, needs_layout_passes=False),
      scratch_types=[
          pltpu.VMEM((CHUNK,), jnp.int32),
          [pltpu.VMEM((IDX_W,), jnp.int32)] * GATHERS_PER_ITER,
          pltpu.VMEM((CHUNK, EMBED_DIM), jnp.float32),
          pltpu.SemaphoreType.DMA,
      ],
  )
  def gather_kernel(ids_hbm, table_hbm, emb_hbm, raw_v, idx_vs, rows_v, sem):
    wid = lax.axis_index("s") * NUM_CORES + lax.axis_index("c")
    blk0 = wid * blocks_per_worker
    row0 = wid * per_worker

    def body(t, carry):
      blk = blk0 + t // CHUNKS_PER_BLK
      sub = t % CHUNKS_PER_BLK
      l_off = blk * TC_BLK + (CHUNK // 2) * sub
      pltpu.sync_copy(ids_hbm.at[pl.ds(l_off, CHUNK // 2)],
                      raw_v.at[pl.ds(0, CHUNK // 2)])
      pltpu.sync_copy(ids_hbm.at[pl.ds(l_off + HALF, CHUNK // 2)],
                      raw_v.at[pl.ds(CHUNK // 2, CHUNK // 2)])
      lane2 = 2 * jnp.arange(LANES, dtype=jnp.int32)
      for v in range(2 * LANES):
        vals = raw_v[pl.ds(LANES * v, LANES)]
        vv = v % LANES
        dst = lane2 + (32 * (vv % 4) + (0 if v < LANES else 1))
        plsc.store_scatter(idx_vs[vv // 4], [dst], vals)
      copies = []
      for j in range(GATHERS_PER_ITER):
        copies.append(
            pltpu.async_copy(
                table_hbm.at[idx_vs[j]],
                rows_v.at[pl.ds(j * IDX_W, IDX_W)],
                sem))
      for c in copies:
        c.wait()
      pltpu.sync_copy(rows_v, emb_hbm.at[pl.ds(row0 + t * CHUNK, CHUNK)])
      return carry

    lax.fori_loop(0, iters, body, 0)

  return gather_kernel(ids1d, table)


CHUNK_BLOCKS = (2, 4, 5, 7, 7)


def _tc_project_chunk(emb2, wt, prev_out, n_rows, blk0, chunk_rows):
  grid = (chunk_rows // TC_BLK,)

  def matmul_kernel(emb_ref, wt_ref, *refs):
    out_ref = refs[-1]
    blk = emb_ref[...]
    out_ref[0:HALF, :] = jnp.dot(
        blk[:, 0:EMBED_DIM], wt_ref[...], preferred_element_type=jnp.float32)
    out_ref[HALF:TC_BLK, :] = jnp.dot(
        blk[:, EMBED_DIM:2 * EMBED_DIM], wt_ref[...],
        preferred_element_type=jnp.float32)

  in_specs = [
      pl.BlockSpec((HALF, 2 * EMBED_DIM), lambda i: (i, 0)),
      pl.BlockSpec((EMBED_DIM, PROJ_DIM), lambda i: (0, 0)),
  ]
  args = [emb2, wt]
  aliases = {}
  if prev_out is not None:
    in_specs.append(pl.BlockSpec(memory_space=pl.ANY))
    args.append(prev_out)
    aliases = {2: 0}
  return pl.pallas_call(
      matmul_kernel,
      grid=grid,
      in_specs=in_specs,
      out_specs=pl.BlockSpec((TC_BLK, PROJ_DIM), lambda i: (blk0 + i, 0)),
      out_shape=jax.ShapeDtypeStruct((n_rows, PROJ_DIM), jnp.float32),
      input_output_aliases=aliases,
  )(*args)


@jax.jit
def _run(token_ids, embed_table, proj_weight):
  b, l = token_ids.shape
  n = b * l
  ids1d = token_ids.astype(jnp.int32).reshape(n)
  wt = proj_weight.T
  out = None
  row0 = 0
  blk0 = 0
  for ub in CHUNK_BLOCKS:
    chunk_rows = ub * TC_BLK * NUM_WORKERS
    ids_c = lax.slice(ids1d, (row0,), (row0 + chunk_rows,))
    emb = _sc_gather(ids_c, embed_table, chunk_rows)
    emb2 = emb.reshape(chunk_rows // 2, 2 * EMBED_DIM)
    out = _tc_project_chunk(emb2, wt, out, n, blk0, chunk_rows)
    row0 += chunk_rows
    blk0 += chunk_rows // TC_BLK
  return out.reshape(b, l, PROJ_DIM)


def kernel(token_ids, embed_table, proj_weight):
  return _run(token_ids, embed_table, proj_weight)


# speedup vs baseline: 1.5040x; 1.0080x over previous
import functools

import jax
import jax.numpy as jnp
from jax import lax
from jax.experimental import pallas as pl
from jax.experimental.pallas import tpu as pltpu
from jax.experimental.pallas import tpu_sc as plsc

NUM_CORES = 2
NUM_SUBCORES = 16
NUM_WORKERS = NUM_CORES * NUM_SUBCORES

EMBED_DIM = 64
PROJ_DIM = 256
LANES = 16

TC_BLK = 4096
HALF = TC_BLK // 2

IDX_W = 128
GATHERS_PER_ITER = 4
CHUNK = IDX_W * GATHERS_PER_ITER
CHUNKS_PER_BLK = TC_BLK // CHUNK


def _sc_gather(ids1d, table, n_rows):
  per_worker = n_rows // NUM_WORKERS
  iters = per_worker // CHUNK
  blocks_per_worker = per_worker // TC_BLK

  mesh = plsc.VectorSubcoreMesh(core_axis_name="c", subcore_axis_name="s")

  @functools.partial(
      pl.kernel,
      mesh=mesh,
      out_type=jax.ShapeDtypeStruct((n_rows, EMBED_DIM), jnp.float32),
      compiler_params=pltpu.CompilerParams(use_tc_tiling_on_sc=False, needs_layout_passes=False),
      scratch_types=[
          pltpu.VMEM((CHUNK,), jnp.int32),
          [pltpu.VMEM((IDX_W,), jnp.int32)] * GATHERS_PER_ITER,
          pltpu.VMEM((CHUNK, EMBED_DIM), jnp.float32),
          pltpu.SemaphoreType.DMA,
      ],
  )
  def gather_kernel(ids_hbm, table_hbm, emb_hbm, raw_v, idx_vs, rows_v, sem):
    wid = lax.axis_index("s") * NUM_CORES + lax.axis_index("c")
    blk0 = wid * blocks_per_worker
    row0 = wid * per_worker

    def body(t, carry):
      blk = blk0 + t // CHUNKS_PER_BLK
      sub = t % CHUNKS_PER_BLK
      l_off = blk * TC_BLK + (CHUNK // 2) * sub
      pltpu.sync_copy(ids_hbm.at[pl.ds(l_off, CHUNK // 2)],
                      raw_v.at[pl.ds(0, CHUNK // 2)])
      pltpu.sync_copy(ids_hbm.at[pl.ds(l_off + HALF, CHUNK // 2)],
                      raw_v.at[pl.ds(CHUNK // 2, CHUNK // 2)])
      lane2 = 2 * jnp.arange(LANES, dtype=jnp.int32)
      for v in range(2 * LANES):
        vals = raw_v[pl.ds(LANES * v, LANES)]
        vv = v % LANES
        dst = lane2 + (32 * (vv % 4) + (0 if v < LANES else 1))
        plsc.store_scatter(idx_vs[vv // 4], [dst], vals)
      copies = []
      for j in range(GATHERS_PER_ITER):
        copies.append(
            pltpu.async_copy(
                table_hbm.at[idx_vs[j]],
                rows_v.at[pl.ds(j * IDX_W, IDX_W)],
                sem))
      for c in copies:
        c.wait()
      pltpu.sync_copy(rows_v, emb_hbm.at[pl.ds(row0 + t * CHUNK, CHUNK)])
      return carry

    lax.fori_loop(0, iters, body, 0)

  return gather_kernel(ids1d, table)


CHUNK_BLOCKS = (1, 2, 4, 6, 6, 6)


def _tc_project_chunk(emb2, wt, prev_out, n_rows, blk0, chunk_rows):
  grid = (chunk_rows // TC_BLK,)

  def matmul_kernel(emb_ref, wt_ref, *refs):
    out_ref = refs[-1]
    blk = emb_ref[...]
    out_ref[0:HALF, :] = jnp.dot(
        blk[:, 0:EMBED_DIM], wt_ref[...], preferred_element_type=jnp.float32)
    out_ref[HALF:TC_BLK, :] = jnp.dot(
        blk[:, EMBED_DIM:2 * EMBED_DIM], wt_ref[...],
        preferred_element_type=jnp.float32)

  in_specs = [
      pl.BlockSpec((HALF, 2 * EMBED_DIM), lambda i: (i, 0)),
      pl.BlockSpec((EMBED_DIM, PROJ_DIM), lambda i: (0, 0)),
  ]
  args = [emb2, wt]
  aliases = {}
  if prev_out is not None:
    in_specs.append(pl.BlockSpec(memory_space=pl.ANY))
    args.append(prev_out)
    aliases = {2: 0}
  return pl.pallas_call(
      matmul_kernel,
      grid=grid,
      in_specs=in_specs,
      out_specs=pl.BlockSpec((TC_BLK, PROJ_DIM), lambda i: (blk0 + i, 0)),
      out_shape=jax.ShapeDtypeStruct((n_rows, PROJ_DIM), jnp.float32),
      input_output_aliases=aliases,
  )(*args)


@jax.jit
def _run(token_ids, embed_table, proj_weight):
  b, l = token_ids.shape
  n = b * l
  ids1d = token_ids.astype(jnp.int32).reshape(n)
  wt = proj_weight.T
  out = None
  row0 = 0
  blk0 = 0
  for ub in CHUNK_BLOCKS:
    chunk_rows = ub * TC_BLK * NUM_WORKERS
    ids_c = lax.slice(ids1d, (row0,), (row0 + chunk_rows,))
    emb = _sc_gather(ids_c, embed_table, chunk_rows)
    emb2 = emb.reshape(chunk_rows // 2, 2 * EMBED_DIM)
    out = _tc_project_chunk(emb2, wt, out, n, blk0, chunk_rows)
    row0 += chunk_rows
    blk0 += chunk_rows // TC_BLK
  return out.reshape(b, l, PROJ_DIM)


def kernel(token_ids, embed_table, proj_weight):
  return _run(token_ids, embed_table, proj_weight)
